# baseline (speedup 1.0000x reference)
import functools

import jax
import jax.numpy as jnp
from jax import lax
from jax.experimental import pallas as pl
from jax.experimental.pallas import tpu as pltpu
from jax.experimental.pallas import tpu_sc as plsc

N = 10000
E = 320000
DF = 128
DE = 16
D = 64
NL = 24
B = 16
S = NL * B

NW = 32
EW = E // NW
C = 80
NCHUNK = EW // C
ACCW = 128
ROWS = 632
NPAD = 16 * ROWS


def _pre_body(x_ref, wn_ref, bn_ref, wq_ref, bq_ref, wk_ref, bk_ref,
              wv_ref, bv_ref, ws_ref, bs_ref,
              q_ref, kv_ref, skip_ref):
    h = jnp.dot(x_ref[...], wn_ref[...],
                preferred_element_type=jnp.float32) + bn_ref[...]
    q_ref[:, :D] = jnp.dot(h, wq_ref[...], preferred_element_type=jnp.float32) + bq_ref[...]
    q_ref[:, D:] = jnp.zeros_like(q_ref[:, D:])
    kv_ref[:, :D] = jnp.dot(h, wk_ref[...], preferred_element_type=jnp.float32) + bk_ref[...]
    kv_ref[:, D:] = jnp.dot(h, wv_ref[...], preferred_element_type=jnp.float32) + bv_ref[...]
    skip_ref[...] = h + jnp.dot(h, ws_ref[...], preferred_element_type=jnp.float32) + bs_ref[...]


def _ee_body(ea_ref, we_ref, be_ref, wd_ref, bd_ref, ee_ref):
    e = jnp.dot(ea_ref[...], we_ref[...],
                preferred_element_type=jnp.float32) + be_ref[...]
    ee_ref[...] = jnp.dot(e, wd_ref[...],
                          preferred_element_type=jnp.float32) + bd_ref[...]


def _dense_pre(x_nodes, edge_attr, Wn, bn, Wq, bq, Wk, bk, Wv, bv, Ws, bs,
               We, be, Wed, bed):
    nb = 1000
    full = lambda shape: pl.BlockSpec(shape, lambda i: (0,) * len(shape))
    q, kv, skip = pl.pallas_call(
        _pre_body,
        grid=(N // nb,),
        in_specs=[pl.BlockSpec((nb, DF), lambda i: (i, 0)),
                  full((DF, D)), full((D,)),
                  full((D, D)), full((D,)), full((D, D)), full((D,)),
                  full((D, D)), full((D,)), full((D, D)), full((D,))],
        out_specs=[pl.BlockSpec((nb, 2 * D), lambda i: (i, 0)),
                   pl.BlockSpec((nb, 2 * D), lambda i: (i, 0)),
                   pl.BlockSpec((nb, D), lambda i: (i, 0))],
        out_shape=[jax.ShapeDtypeStruct((N, 2 * D), jnp.float32),
                   jax.ShapeDtypeStruct((N, 2 * D), jnp.float32),
                   jax.ShapeDtypeStruct((N, D), jnp.float32)],
    )(x_nodes, Wn, bn, Wq, bq, Wk, bk, Wv, bv, Ws, bs)

    ebk = 4000
    ea2 = edge_attr.reshape(E // 2, 2 * DE)
    We2 = jnp.zeros((2 * DE, 2 * D), We.dtype)
    We2 = We2.at[:DE, :D].set(We).at[DE:, D:].set(We)
    be2 = jnp.concatenate([be, be])
    Wed2 = jnp.zeros((2 * D, 2 * D), Wed.dtype)
    Wed2 = Wed2.at[:D, :D].set(Wed).at[D:, D:].set(Wed)
    bed2 = jnp.concatenate([bed, bed])
    ee2 = pl.pallas_call(
        _ee_body,
        grid=(E // 2 // ebk,),
        in_specs=[pl.BlockSpec((ebk, 2 * DE), lambda i: (i, 0)),
                  full((2 * DE, 2 * D)), full((2 * D,)),
                  full((2 * D, 2 * D)), full((2 * D,))],
        out_specs=pl.BlockSpec((ebk, 2 * D), lambda i: (i, 0)),
        out_shape=jax.ShapeDtypeStruct((E // 2, 2 * D), jnp.float32),
    )(ea2, We2, be2, Wed2, bed2)
    return q, kv, skip, ee2


CF = 80
NCH = EW // CF


def _edge_sc_body(src_hbm, dst_hbm, q_hbm, kv_hbm, ee_hbm, out_hbm,
                  sidx, didx, qb, kvb, eb, mb, acc_sh, semg, semi):
    c = lax.axis_index("c")
    s = lax.axis_index("s")
    wid = s * 2 + c
    base = s * ROWS
    e0 = wid * EW

    z16 = jnp.zeros((16,), jnp.float32)

    def zrow(j, carry):
        for t in range(ACCW // 16):
            mb[j, pl.ds(16 * t, 16)] = z16
        return carry

    lax.fori_loop(0, CF, zrow, 0)
    for ofs, ln in ((0, 80), (80, 80), (160, 80), (240, 80), (320, 80),
                    (400, 80), (480, 80), (560, 72)):
        pltpu.sync_copy(mb.at[pl.ds(0, ln)], acc_sh.at[pl.ds(base + ofs, ln)])
    plsc.subcore_barrier()

    lane0 = jnp.where(lax.iota(jnp.int32, 16) == 0, 1.0, 0.0)
    _GDN = lax.GatherDimensionNumbers(offset_dims=(), collapsed_slice_dims=(0,),
                                      start_index_map=(0,))
    lanes = lax.iota(jnp.int32, 16)
    perms = [(lanes ^ sh)[:, None] for sh in (8, 4, 2, 1)]

    def do_edge(ei, er, ec):
        acc = jnp.zeros((16,), jnp.float32)
        evs = []
        for t in range(4):
            sl = pl.ds(16 * t, 16)
            ev = eb[er, pl.ds(ec + 16 * t, 16)]
            evs.append(ev)
            acc = acc + qb[ei, sl] * (kvb[ei, sl] + ev)
        for p in perms:
            acc = acc + lax.gather(acc, p, _GDN, (1,),
                                   mode=lax.GatherScatterMode.PROMISE_IN_BOUNDS)
        w = jnp.exp(acc * 0.125)
        for t in range(4):
            mb[ei, pl.ds(16 * t, 16)] = w * (kvb[ei, pl.ds(D + 16 * t, 16)] + evs[t])
        mb[ei, pl.ds(64, 16)] = w * lane0

    def chunk(i, carry):
        off = pl.multiple_of(e0 + i * CF, 16)
        h1 = pltpu.async_copy(src_hbm.at[pl.ds(off, CF)], sidx, semi)
        h2 = pltpu.async_copy(dst_hbm.at[pl.ds(off, CF)], didx, semi)
        eoff = pl.multiple_of(off // 2, 8)
        h3 = pltpu.async_copy(ee_hbm.at[pl.ds(eoff, CF // 2)], eb, semi)
        h1.wait()
        h2.wait()
        g1 = pltpu.async_copy(q_hbm.at[didx], qb, semg)
        g2 = pltpu.async_copy(kv_hbm.at[sidx], kvb, semg)
        h3.wait()
        g1.wait()
        g2.wait()

        def oct8(j, carry2):
            b8 = j * 8
            r4 = j * 4
            for u in range(8):
                do_edge(b8 + u, r4 + u // 2, (u % 2) * D)
            return carry2

        lax.fori_loop(0, CF // 8, oct8, 0)
        pltpu.sync_copy(mb, acc_sh.at[didx], add=True)
        return carry

    lax.fori_loop(0, NCH, chunk, 0)
    plsc.subcore_barrier()
    pltpu.sync_copy(acc_sh.at[pl.ds(base, ROWS)],
                    out_hbm.at[c, pl.ds(base, ROWS)])


def _edge_sc(src, dst, q, kv, ee2):
    mesh = plsc.VectorSubcoreMesh(core_axis_name="c", subcore_axis_name="s")
    f = functools.partial(
        pl.kernel, _edge_sc_body, mesh=mesh,
        out_type=jax.ShapeDtypeStruct((2, NPAD, ACCW), jnp.float32),
        scratch_types=[
            pltpu.VMEM((CF,), jnp.int32),
            pltpu.VMEM((CF,), jnp.int32),
            pltpu.VMEM((CF, 2 * D), jnp.float32),
            pltpu.VMEM((CF, 2 * D), jnp.float32),
            pltpu.VMEM((CF // 2, 2 * D), jnp.float32),
            pltpu.VMEM((CF, ACCW), jnp.float32),
            pltpu.VMEM_SHARED((NPAD, ACCW), jnp.float32),
            pltpu.SemaphoreType.DMA,
            pltpu.SemaphoreType.DMA,
        ],
    )()
    return f(src, dst, q, kv, ee2)


def _post_body(a0_ref, a1_ref, skip_ref, seg_ref, w1_ref, b1_ref,
               w2_ref, b2_ref, y_ref, pool_ref, cnt_ref):
    i = pl.program_id(0)
    nb = skip_ref.shape[0]
    num = a0_ref[:, :D] + a1_ref[:, :D]
    den = a0_ref[:, D:D + 1] + a1_ref[:, D:D + 1]
    out = num / (den + 1e-16) + skip_ref[...]
    out = jnp.maximum(out, 0.0)
    seg = seg_ref[...]
    sids = lax.broadcasted_iota(jnp.int32, (nb, S), 1)
    onehot = (sids == seg).astype(jnp.float32)

    @pl.when(i == 0)
    def _():
        pool_ref[...] = jnp.zeros_like(pool_ref)
        cnt_ref[...] = jnp.zeros_like(cnt_ref)

    pool_ref[...] += lax.dot_general(onehot, out, (((0,), (0,)), ((), ())),
                                     preferred_element_type=jnp.float32, precision=lax.Precision.HIGHEST)
    cnt_ref[...] += lax.dot_general(onehot, jnp.ones((nb, 1), jnp.float32),
                                    (((0,), (0,)), ((), ())),
                                    preferred_element_type=jnp.float32, precision=lax.Precision.HIGHEST)

    @pl.when(i == pl.num_programs(0) - 1)
    def _():
        g = pool_ref[...] / jnp.maximum(cnt_ref[...], 1.0)
        g = jnp.maximum(jnp.dot(g, w1_ref[...],
                                preferred_element_type=jnp.float32) + b1_ref[...], 0.0)
        y_ref[...] = jnp.dot(g, w2_ref[...],
                             preferred_element_type=jnp.float32) + b2_ref[...]


def _post(a0, a1, skip, seg, W1, b1, W2, b2):
    nb = 1000
    full = lambda shape: pl.BlockSpec(shape, lambda i: (0,) * len(shape))
    return pl.pallas_call(
        _post_body,
        grid=(N // nb,),
        in_specs=[pl.BlockSpec((nb, ACCW), lambda i: (i, 0)),
                  pl.BlockSpec((nb, ACCW), lambda i: (i, 0)),
                  pl.BlockSpec((nb, D), lambda i: (i, 0)),
                  pl.BlockSpec((nb, 1), lambda i: (i, 0)),
                  full((D, 2 * D)), full((2 * D,)),
                  full((2 * D, 1)), full((1,))],
        out_specs=full((S, 1)),
        out_shape=jax.ShapeDtypeStruct((S, 1), jnp.float32),
        scratch_shapes=[pltpu.VMEM((S, D), jnp.float32),
                        pltpu.VMEM((S, 1), jnp.float32)],
    )(a0, a1, skip, seg, W1, b1, W2, b2)


def kernel(x_nodes, edge_index, edge_attr, location, batch,
           Wn, bn, We, be, Wq, bq, Wk, bk, Wv, bv, Wed, bed, Ws, bs,
           W1, b1, W2, b2):
    q, kv, skip, ee2 = _dense_pre(x_nodes, edge_attr, Wn, bn, Wq, bq,
                                  Wk, bk, Wv, bv, Ws, bs, We, be, Wed, bed)
    acc = _edge_sc(edge_index[0], edge_index[1], q, kv, ee2)
    seg = (location + NL * batch).astype(jnp.int32).reshape(N, 1)
    return _post(acc[0], acc[1], skip, seg, W1, b1, W2, b2)


# speedup vs baseline: 1.0394x; 1.0394x over previous
import functools

import jax
import jax.numpy as jnp
from jax import lax
from jax.experimental import pallas as pl
from jax.experimental.pallas import tpu as pltpu
from jax.experimental.pallas import tpu_sc as plsc

N = 10000
E = 320000
DF = 128
DE = 16
D = 64
NL = 24
B = 16
S = NL * B

NW = 32
EW = E // NW
C = 80
NCHUNK = EW // C
ACCW = 128
ROWS = 632
NPAD = 16 * ROWS


def _pre_body(x_ref, wn_ref, bn_ref, wq_ref, bq_ref, wk_ref, bk_ref,
              wv_ref, bv_ref, ws_ref, bs_ref,
              q_ref, kv_ref, skip_ref):
    h = jnp.dot(x_ref[...], wn_ref[...],
                preferred_element_type=jnp.float32) + bn_ref[...]
    q_ref[:, :D] = jnp.dot(h, wq_ref[...], preferred_element_type=jnp.float32) + bq_ref[...]
    q_ref[:, D:] = jnp.zeros_like(q_ref[:, D:])
    kv_ref[:, :D] = jnp.dot(h, wk_ref[...], preferred_element_type=jnp.float32) + bk_ref[...]
    kv_ref[:, D:] = jnp.dot(h, wv_ref[...], preferred_element_type=jnp.float32) + bv_ref[...]
    skip_ref[...] = h + jnp.dot(h, ws_ref[...], preferred_element_type=jnp.float32) + bs_ref[...]


def _ee_body(ea_ref, we_ref, be_ref, wd_ref, bd_ref, ee_ref):
    e = jnp.dot(ea_ref[...], we_ref[...],
                preferred_element_type=jnp.float32) + be_ref[...]
    ee_ref[...] = jnp.dot(e, wd_ref[...],
                          preferred_element_type=jnp.float32) + bd_ref[...]


def _dense_pre(x_nodes, edge_attr, Wn, bn, Wq, bq, Wk, bk, Wv, bv, Ws, bs,
               We, be, Wed, bed):
    nb = 1000
    full = lambda shape: pl.BlockSpec(shape, lambda i: (0,) * len(shape))
    q, kv, skip = pl.pallas_call(
        _pre_body,
        grid=(N // nb,),
        in_specs=[pl.BlockSpec((nb, DF), lambda i: (i, 0)),
                  full((DF, D)), full((D,)),
                  full((D, D)), full((D,)), full((D, D)), full((D,)),
                  full((D, D)), full((D,)), full((D, D)), full((D,))],
        out_specs=[pl.BlockSpec((nb, 2 * D), lambda i: (i, 0)),
                   pl.BlockSpec((nb, 2 * D), lambda i: (i, 0)),
                   pl.BlockSpec((nb, D), lambda i: (i, 0))],
        out_shape=[jax.ShapeDtypeStruct((N, 2 * D), jnp.float32),
                   jax.ShapeDtypeStruct((N, 2 * D), jnp.float32),
                   jax.ShapeDtypeStruct((N, D), jnp.float32)],
    )(x_nodes, Wn, bn, Wq, bq, Wk, bk, Wv, bv, Ws, bs)

    ebk = 4000
    ea2 = edge_attr.reshape(E // 2, 2 * DE)
    We2 = jnp.zeros((2 * DE, 2 * D), We.dtype)
    We2 = We2.at[:DE, :D].set(We).at[DE:, D:].set(We)
    be2 = jnp.concatenate([be, be])
    Wed2 = jnp.zeros((2 * D, 2 * D), Wed.dtype)
    Wed2 = Wed2.at[:D, :D].set(Wed).at[D:, D:].set(Wed)
    bed2 = jnp.concatenate([bed, bed])
    ee2 = pl.pallas_call(
        _ee_body,
        grid=(E // 2 // ebk,),
        in_specs=[pl.BlockSpec((ebk, 2 * DE), lambda i: (i, 0)),
                  full((2 * DE, 2 * D)), full((2 * D,)),
                  full((2 * D, 2 * D)), full((2 * D,))],
        out_specs=pl.BlockSpec((ebk, 2 * D), lambda i: (i, 0)),
        out_shape=jax.ShapeDtypeStruct((E // 2, 2 * D), jnp.float32),
    )(ea2, We2, be2, Wed2, bed2)
    return q, kv, skip, ee2


CF = 80
NCH = EW // CF


def _edge_sc_body(src_hbm, dst_hbm, q_hbm, kv_hbm, ee_hbm, out_hbm,
                  sidxA, sidxB, didxA, didxB, qb, kvb, ebA, ebB, mb,
                  acc_sh, semg, semiA, semiB):
    c = lax.axis_index("c")
    s = lax.axis_index("s")
    wid = s * 2 + c
    base = s * ROWS
    e0 = wid * EW

    z16 = jnp.zeros((16,), jnp.float32)

    def zrow(j, carry):
        for t in range(ACCW // 16):
            mb[j, pl.ds(16 * t, 16)] = z16
        return carry

    lax.fori_loop(0, CF, zrow, 0)
    for ofs, ln in ((0, 80), (80, 80), (160, 80), (240, 80), (320, 80),
                    (400, 80), (480, 80), (560, 72)):
        pltpu.sync_copy(mb.at[pl.ds(0, ln)], acc_sh.at[pl.ds(base + ofs, ln)])
    plsc.subcore_barrier()

    lane0 = jnp.where(lax.iota(jnp.int32, 16) == 0, 1.0, 0.0)
    _GDN = lax.GatherDimensionNumbers(offset_dims=(), collapsed_slice_dims=(0,),
                                      start_index_map=(0,))
    lanes = lax.iota(jnp.int32, 16)
    perms = [(lanes ^ sh)[:, None] for sh in (8, 4, 2, 1)]

    def do_edge(ei, er, ec, eb):
        acc = jnp.zeros((16,), jnp.float32)
        evs = []
        for t in range(4):
            sl = pl.ds(16 * t, 16)
            ev = eb[er, pl.ds(ec + 16 * t, 16)]
            evs.append(ev)
            acc = acc + qb[ei, sl] * (kvb[ei, sl] + ev)
        for p in perms:
            acc = acc + lax.gather(acc, p, _GDN, (1,),
                                   mode=lax.GatherScatterMode.PROMISE_IN_BOUNDS)
        w = jnp.exp(acc * 0.125)
        for t in range(4):
            mb[ei, pl.ds(16 * t, 16)] = w * (kvb[ei, pl.ds(D + 16 * t, 16)] + evs[t])
        mb[ei, pl.ds(64, 16)] = w * lane0

    def proc(ch, sidx, didx, eb, semi):
        off = pl.multiple_of(e0 + ch * CF, 16)
        eoff = pl.multiple_of(off // 2, 8)
        pltpu.make_async_copy(src_hbm.at[pl.ds(off, CF)], sidx, semi).wait()
        pltpu.make_async_copy(dst_hbm.at[pl.ds(off, CF)], didx, semi).wait()
        pltpu.make_async_copy(ee_hbm.at[pl.ds(eoff, CF // 2)], eb, semi).wait()
        g1 = pltpu.async_copy(q_hbm.at[didx], qb, semg)
        g2 = pltpu.async_copy(kv_hbm.at[sidx], kvb, semg)
        g1.wait()
        g2.wait()

        def oct8(j, carry2):
            b8 = j * 8
            r4 = j * 4
            for u in range(8):
                do_edge(b8 + u, r4 + u // 2, (u % 2) * D, eb)
            return carry2

        lax.fori_loop(0, CF // 8, oct8, 0)
        pltpu.sync_copy(mb, acc_sh.at[didx], add=True)
        ch2 = jnp.minimum(ch + 2, NCH - 1)
        off2 = pl.multiple_of(e0 + ch2 * CF, 16)
        eoff2 = pl.multiple_of(off2 // 2, 8)
        pltpu.async_copy(src_hbm.at[pl.ds(off2, CF)], sidx, semi)
        pltpu.async_copy(dst_hbm.at[pl.ds(off2, CF)], didx, semi)
        pltpu.async_copy(ee_hbm.at[pl.ds(eoff2, CF // 2)], eb, semi)

    def startup(ch, sidx, didx, eb, semi):
        off = pl.multiple_of(e0 + ch * CF, 16)
        eoff = pl.multiple_of(off // 2, 8)
        pltpu.async_copy(src_hbm.at[pl.ds(off, CF)], sidx, semi)
        pltpu.async_copy(dst_hbm.at[pl.ds(off, CF)], didx, semi)
        pltpu.async_copy(ee_hbm.at[pl.ds(eoff, CF // 2)], eb, semi)

    startup(0, sidxA, didxA, ebA, semiA)
    startup(1, sidxB, didxB, ebB, semiB)

    def pair(tp, carry):
        proc(tp * 2, sidxA, didxA, ebA, semiA)
        proc(tp * 2 + 1, sidxB, didxB, ebB, semiB)
        return carry

    lax.fori_loop(0, NCH // 2, pair, 0)
    proc(NCH - 1, sidxA, didxA, ebA, semiA)
    for sidx, didx, eb, semi in ((sidxA, didxA, ebA, semiA),
                                 (sidxB, didxB, ebB, semiB)):
        offl = pl.multiple_of(e0 + (NCH - 1) * CF, 16)
        eoffl = pl.multiple_of(offl // 2, 8)
        pltpu.make_async_copy(src_hbm.at[pl.ds(offl, CF)], sidx, semi).wait()
        pltpu.make_async_copy(dst_hbm.at[pl.ds(offl, CF)], didx, semi).wait()
        pltpu.make_async_copy(ee_hbm.at[pl.ds(eoffl, CF // 2)], eb, semi).wait()

    plsc.subcore_barrier()
    pltpu.sync_copy(acc_sh.at[pl.ds(base, ROWS)],
                    out_hbm.at[c, pl.ds(base, ROWS)])


def _edge_sc(src, dst, q, kv, ee2):
    mesh = plsc.VectorSubcoreMesh(core_axis_name="c", subcore_axis_name="s")
    f = functools.partial(
        pl.kernel, _edge_sc_body, mesh=mesh,
        out_type=jax.ShapeDtypeStruct((2, NPAD, ACCW), jnp.float32),
        scratch_types=[
            pltpu.VMEM((CF,), jnp.int32),
            pltpu.VMEM((CF,), jnp.int32),
            pltpu.VMEM((CF,), jnp.int32),
            pltpu.VMEM((CF,), jnp.int32),
            pltpu.VMEM((CF, 2 * D), jnp.float32),
            pltpu.VMEM((CF, 2 * D), jnp.float32),
            pltpu.VMEM((CF // 2, 2 * D), jnp.float32),
            pltpu.VMEM((CF // 2, 2 * D), jnp.float32),
            pltpu.VMEM((CF, ACCW), jnp.float32),
            pltpu.VMEM_SHARED((NPAD, ACCW), jnp.float32),
            pltpu.SemaphoreType.DMA,
            pltpu.SemaphoreType.DMA,
            pltpu.SemaphoreType.DMA,
        ],
    )()
    return f(src, dst, q, kv, ee2)


def _post_body(a0_ref, a1_ref, skip_ref, seg_ref, w1_ref, b1_ref,
               w2_ref, b2_ref, y_ref, pool_ref, cnt_ref):
    i = pl.program_id(0)
    nb = skip_ref.shape[0]
    num = a0_ref[:, :D] + a1_ref[:, :D]
    den = a0_ref[:, D:D + 1] + a1_ref[:, D:D + 1]
    out = num / (den + 1e-16) + skip_ref[...]
    out = jnp.maximum(out, 0.0)
    seg = seg_ref[...]
    sids = lax.broadcasted_iota(jnp.int32, (nb, S), 1)
    onehot = (sids == seg).astype(jnp.float32)

    @pl.when(i == 0)
    def _():
        pool_ref[...] = jnp.zeros_like(pool_ref)
        cnt_ref[...] = jnp.zeros_like(cnt_ref)

    pool_ref[...] += lax.dot_general(onehot, out, (((0,), (0,)), ((), ())),
                                     preferred_element_type=jnp.float32, precision=lax.Precision.HIGHEST)
    cnt_ref[...] += lax.dot_general(onehot, jnp.ones((nb, 1), jnp.float32),
                                    (((0,), (0,)), ((), ())),
                                    preferred_element_type=jnp.float32, precision=lax.Precision.HIGHEST)

    @pl.when(i == pl.num_programs(0) - 1)
    def _():
        g = pool_ref[...] / jnp.maximum(cnt_ref[...], 1.0)
        g = jnp.maximum(jnp.dot(g, w1_ref[...],
                                preferred_element_type=jnp.float32) + b1_ref[...], 0.0)
        y_ref[...] = jnp.dot(g, w2_ref[...],
                             preferred_element_type=jnp.float32) + b2_ref[...]


def _post(a0, a1, skip, seg, W1, b1, W2, b2):
    nb = 1000
    full = lambda shape: pl.BlockSpec(shape, lambda i: (0,) * len(shape))
    return pl.pallas_call(
        _post_body,
        grid=(N // nb,),
        in_specs=[pl.BlockSpec((nb, ACCW), lambda i: (i, 0)),
                  pl.BlockSpec((nb, ACCW), lambda i: (i, 0)),
                  pl.BlockSpec((nb, D), lambda i: (i, 0)),
                  pl.BlockSpec((nb, 1), lambda i: (i, 0)),
                  full((D, 2 * D)), full((2 * D,)),
                  full((2 * D, 1)), full((1,))],
        out_specs=full((S, 1)),
        out_shape=jax.ShapeDtypeStruct((S, 1), jnp.float32),
        scratch_shapes=[pltpu.VMEM((S, D), jnp.float32),
                        pltpu.VMEM((S, 1), jnp.float32)],
    )(a0, a1, skip, seg, W1, b1, W2, b2)


def kernel(x_nodes, edge_index, edge_attr, location, batch,
           Wn, bn, We, be, Wq, bq, Wk, bk, Wv, bv, Wed, bed, Ws, bs,
           W1, b1, W2, b2):
    q, kv, skip, ee2 = _dense_pre(x_nodes, edge_attr, Wn, bn, Wq, bq,
                                  Wk, bk, Wv, bv, Ws, bs, We, be, Wed, bed)
    acc = _edge_sc(edge_index[0], edge_index[1], q, kv, ee2)
    seg = (location + NL * batch).astype(jnp.int32).reshape(N, 1)
    return _post(acc[0], acc[1], skip, seg, W1, b1, W2, b2)
